# baseline (speedup 1.0000x reference)
import functools

import jax
import jax.numpy as jnp
from jax import lax
from jax.experimental import pallas as pl
from jax.experimental.pallas import tpu as pltpu
from jax.experimental.pallas import tpu_sc as plsc

N = 10000
E = 320000
MAX_NB = 12
H = 300
HP = 304

NW = 32
NPAD = 10240
A_W = NPAD // NW
A_C = 8
B_W = E // NW
B_C = 80

_mesh = plsc.VectorSubcoreMesh(core_axis_name="c", subcore_axis_name="s")
_sc_params = pltpu.CompilerParams(use_tc_tiling_on_sc=False)


def _mm(x, w, *, bm, relu_in=False, res=None, bias=None, relu_out=False):
    m, k = x.shape
    _, n = w.shape
    grid = (m // bm,)

    def body(*refs):
        x_ref, w_ref = refs[0], refs[1]
        i = 2
        r_ref = b_ref = None
        if res is not None:
            r_ref = refs[i]; i += 1
        if bias is not None:
            b_ref = refs[i]; i += 1
        o_ref = refs[i]
        xv = x_ref[...]
        if relu_in:
            xv = jnp.maximum(xv, 0.0)
        acc = jnp.dot(xv, w_ref[...], preferred_element_type=jnp.float32)
        if res is not None:
            acc = acc + r_ref[...]
        if bias is not None:
            acc = acc + b_ref[...]
        if relu_out:
            acc = jnp.maximum(acc, 0.0)
        o_ref[...] = acc

    in_specs = [pl.BlockSpec((bm, k), lambda i: (i, 0)),
                pl.BlockSpec((k, n), lambda i: (0, 0))]
    args = [x, w]
    if res is not None:
        in_specs.append(pl.BlockSpec((bm, n), lambda i: (i, 0)))
        args.append(res)
    if bias is not None:
        in_specs.append(pl.BlockSpec((1, n), lambda i: (0, 0)))
        args.append(bias.reshape(1, n))
    return pl.pallas_call(
        body, grid=grid, in_specs=in_specs,
        out_specs=pl.BlockSpec((bm, n), lambda i: (i, 0)),
        out_shape=jax.ShapeDtypeStruct((m, n), jnp.float32),
    )(*args)


def _mm_dual(x, w1, w2, *, bm):
    m, k = x.shape
    n1 = w1.shape[1]
    n2 = w2.shape[1]
    grid = (m // bm,)

    def body(x_ref, w1_ref, w2_ref, o1_ref, o2_ref):
        xv = x_ref[...]
        o1_ref[...] = jnp.dot(xv, w1_ref[...], preferred_element_type=jnp.float32)
        o2_ref[...] = jnp.dot(xv, w2_ref[...], preferred_element_type=jnp.float32)

    return pl.pallas_call(
        body, grid=grid,
        in_specs=[pl.BlockSpec((bm, k), lambda i: (i, 0)),
                  pl.BlockSpec((k, n1), lambda i: (0, 0)),
                  pl.BlockSpec((k, n2), lambda i: (0, 0))],
        out_specs=[pl.BlockSpec((bm, n1), lambda i: (i, 0)),
                   pl.BlockSpec((bm, n2), lambda i: (i, 0))],
        out_shape=[jax.ShapeDtypeStruct((m, n1), jnp.float32),
                   jax.ShapeDtypeStruct((m, n2), jnp.float32)],
    )(x, w1, w2)


@functools.partial(
    pl.kernel,
    out_type=jax.ShapeDtypeStruct((NPAD, HP), jnp.float32),
    mesh=_mesh,
    scratch_types=[
        pltpu.VMEM((A_C * MAX_NB,), jnp.int32),
        pltpu.VMEM((A_C * MAX_NB, HP), jnp.float32),
        pltpu.VMEM((A_C, HP), jnp.float32),
        pltpu.SemaphoreType.DMA,
    ],
    compiler_params=_sc_params,
)
def _segsum(mp_hbm, a2b_hbm, out_hbm, idx_v, rows_v, out_v, sem):
    wid = lax.axis_index("s") * 2 + lax.axis_index("c")

    def chunk(t, carry):
        base_a = pl.multiple_of(wid * A_W + t * A_C, A_C)
        pltpu.sync_copy(a2b_hbm.at[pl.ds(base_a * MAX_NB, A_C * MAX_NB)], idx_v)
        pltpu.async_copy(mp_hbm.at[idx_v], rows_v, sem).wait()

        def atom(a, carry2):
            for c in range(HP // 16):
                sl = pl.ds(c * 16, 16)
                acc = jnp.maximum(rows_v[a * MAX_NB, sl], 0.0)
                for k in range(1, MAX_NB):
                    acc = acc + jnp.maximum(rows_v[a * MAX_NB + k, sl], 0.0)
                out_v[a, sl] = acc
            return carry2

        lax.fori_loop(0, A_C, atom, 0)
        pltpu.sync_copy(out_v, out_hbm.at[pl.ds(base_a, A_C)])
        return carry

    lax.fori_loop(0, A_W // A_C, chunk, 0)


@functools.partial(
    pl.kernel,
    out_type=jax.ShapeDtypeStruct((E, HP), jnp.float32),
    mesh=_mesh,
    scratch_types=[
        pltpu.VMEM((B_C,), jnp.int32),
        pltpu.VMEM((B_C,), jnp.int32),
        pltpu.VMEM((B_C, HP), jnp.float32),
        pltpu.SemaphoreType.DMA,
    ],
    compiler_params=_sc_params,
)
def _update(inp_hbm, u_hbm, vn_hbm, b2a_hbm, b2revb_hbm, out_hbm,
            ia_v, ir_v, buf_v, sem):
    wid = lax.axis_index("s") * 2 + lax.axis_index("c")

    def chunk(t, carry):
        base = pl.multiple_of(wid * B_W + t * B_C, B_C)
        pltpu.sync_copy(inp_hbm.at[pl.ds(base, B_C)], buf_v)
        pltpu.sync_copy(b2a_hbm.at[pl.ds(base, B_C)], ia_v)
        pltpu.sync_copy(b2revb_hbm.at[pl.ds(base, B_C)], ir_v)
        pltpu.async_copy(u_hbm.at[ia_v], buf_v, sem, add=True).wait()
        pltpu.async_copy(vn_hbm.at[ir_v], buf_v, sem, add=True).wait()
        pltpu.sync_copy(buf_v, out_hbm.at[pl.ds(base, B_C)])
        return carry

    lax.fori_loop(0, B_W // B_C, chunk, 0)


def kernel(f_atoms, f_bonds, a2b, b2a, b2revb, W_i, W_h, W_o_a, b_o_a, W_o_b, b_o_b):
    f32 = jnp.float32
    W_i_p = jnp.pad(W_i.astype(f32), ((0, 0), (0, HP - H)))
    W_h_p = jnp.pad(W_h.astype(f32), ((0, HP - H), (0, HP - H)))
    W_h_n = -W_h_p
    Wob_top = W_o_b[:f_bonds.shape[1]].astype(f32)
    Wob_bot = jnp.pad(W_o_b[f_bonds.shape[1]:].astype(f32), ((0, HP - H), (0, 0)))

    a2b_flat = jnp.pad(a2b.astype(jnp.int32), ((0, NPAD - N), (0, 0))).reshape(-1)
    b2a_i = b2a.astype(jnp.int32)
    b2revb_i = b2revb.astype(jnp.int32)

    inp, fbo = _mm_dual(f_bonds, W_i_p, Wob_top, bm=1600)

    mp = inp
    for _ in range(2):
        a_msg = _segsum(mp, a2b_flat)
        u = _mm(a_msg, W_h_p, bm=2048)
        vn = _mm(mp, W_h_n, bm=1600, relu_in=True)
        mp = _update(inp, u, vn, b2a_i, b2revb_i)

    a_msg_f = _segsum(mp, a2b_flat)

    bond_hiddens = _mm(mp, Wob_bot, bm=1600, relu_in=True,
                       res=fbo, bias=b_o_b, relu_out=True)

    a_in = jnp.concatenate([f_atoms, a_msg_f[:N, :H]], axis=1)
    atom_hiddens = _mm(a_in, W_o_a, bm=2000, bias=b_o_a, relu_out=True)

    return atom_hiddens, bond_hiddens
